# Initial kernel scaffold; baseline (speedup 1.0000x reference)
import jax
import jax.numpy as jnp
from jax.experimental import pallas as pl


def kernel(token_ids, weight):
    raise NotImplementedError("write your pallas kernel here")


# speedup vs baseline: 1.8377x; 1.8377x over previous
import functools

import jax
import jax.numpy as jnp
from jax import lax
from jax.experimental import pallas as pl
from jax.experimental.pallas import tpu as pltpu
from jax.experimental.pallas import tpu_sc as plsc

NC = 2
NS = 16
NW = NC * NS

SLICE = 128


@functools.partial(jax.jit, static_argnums=(2, 3))
def _sc_gather(idx2d, weight, n_slices_w, dim):
    n_slices = idx2d.shape[0]
    b_total = n_slices * SLICE
    mesh = plsc.VectorSubcoreMesh(
        core_axis_name="c", subcore_axis_name="s", num_cores=NC, num_subcores=NS
    )

    @functools.partial(
        pl.kernel,
        out_type=jax.ShapeDtypeStruct((b_total, dim), jnp.float32),
        mesh=mesh,
        scratch_types=[
            pltpu.VMEM((n_slices_w, SLICE), jnp.int32),
            pltpu.VMEM((2, SLICE, dim), jnp.float32),
            pltpu.SemaphoreType.DMA,
            pltpu.SemaphoreType.DMA,
        ],
        compiler_params=pltpu.CompilerParams(use_tc_tiling_on_sc=False),
    )
    def k(idx_hbm, table_hbm, out_hbm, idx_v, rows_v, gsem0, gsem1):
        wid = lax.axis_index("s") * NC + lax.axis_index("c")
        srow = wid * n_slices_w
        pltpu.sync_copy(idx_hbm.at[pl.ds(srow, n_slices_w)], idx_v)

        gsems = (gsem0, gsem1)

        def gather(j, b):
            return pltpu.async_copy(
                table_hbm.at[idx_v.at[j]], rows_v.at[b], gsems[b]
            )

        gather(0, 0)
        gather(1, 1)

        def pair(g, _):
            for b in range(2):
                j = g * 2 + b
                pltpu.make_async_copy(
                    table_hbm.at[idx_v.at[j]], rows_v.at[b], gsems[b]
                ).wait()
                pltpu.sync_copy(
                    rows_v.at[b], out_hbm.at[pl.ds((srow + j) * SLICE, SLICE)]
                )

                @pl.when(j + 2 < n_slices_w)
                def _():
                    gather(j + 2, b)

            return 0

        lax.fori_loop(0, n_slices_w // 2, pair, 0, unroll=False)

    return k(idx2d, weight)


def kernel(token_ids, weight):
    b0, s = token_ids.shape
    b = b0 * s
    dim = weight.shape[1]
    n_slices = b // SLICE
    n_slices_w = n_slices // NW
    idx2d = token_ids.reshape(n_slices, SLICE).astype(jnp.int32)
    out = _sc_gather(idx2d, weight, n_slices_w, dim)
    return out.reshape(b0, s, dim)
